# baseline (speedup 1.0000x reference)
import functools

import jax
import jax.numpy as jnp
from jax import lax
from jax.experimental import pallas as pl
from jax.experimental.pallas import tpu as pltpu
from jax.experimental.pallas import tpu_sc as plsc

DIM = 64
GAMMA = 12.0
LANE = 128


PACKW = 129


def _transpose_slab(slab, pack, parity64, row_half, L):
    @plsc.parallel_loop(0, DIM, unroll=16)
    def _(d):
        col = parity64 + d
        for k in range(LANE // L):
            v = slab[d, pl.ds(k * L, L)]
            rows = row_half + 8 * k
            plsc.store_scatter(pack, [rows, col], v)


@functools.cache
def _make_phase_a(n_entity: int):
    info = plsc.get_sparse_core_info()
    NC, NS, L = info.num_cores, info.num_subcores, info.num_lanes
    NW = NC * NS
    NSLAB = (n_entity + LANE - 1) // LANE
    NROW = NSLAB * DIM
    STEPS = (NSLAB + 2 * NW - 1) // (2 * NW)
    mesh = plsc.VectorSubcoreMesh(core_axis_name="c", subcore_axis_name="s")
    stage_t = jax.ShapeDtypeStruct((NROW, 2 * DIM), jnp.float32)

    @functools.partial(
        pl.kernel,
        mesh=mesh,
        compiler_params=pltpu.CompilerParams(needs_layout_passes=False),
        out_type=(stage_t, stage_t),
        scratch_types=[
            pltpu.VMEM((2, DIM, LANE), jnp.float32),
            pltpu.VMEM((2, DIM, LANE), jnp.float32),
            pltpu.VMEM((2, DIM, PACKW), jnp.float32),
            pltpu.VMEM((2, DIM, PACKW), jnp.float32),
            pltpu.SemaphoreType.DMA,
            pltpu.SemaphoreType.DMA,
            pltpu.SemaphoreType.DMA,
            pltpu.SemaphoreType.DMA,
        ],
    )
    def ka(ent_hbm, rel_hbm, eout_hbm, rout_hbm,
           eslab, rslab, epack, rpack, rsem0, rsem1, wsem0, wsem1):
        wid = lax.axis_index("s") * NC + lax.axis_index("c")
        lanes = lax.iota(jnp.int32, L)
        parity64 = (lanes & 1) * DIM
        row_half = lanes >> 1

        def slab_of(t):
            return jnp.minimum(wid + NW * t, NSLAB - 1)

        def fire_reads(t, slot, rsem):
            cb = slab_of(t)
            col = pl.ds(cb * LANE, LANE)
            pltpu.async_copy(ent_hbm.at[:, col], eslab.at[slot], rsem)
            pltpu.async_copy(rel_hbm.at[:, col], rslab.at[slot], rsem)

        dummy = ent_hbm.at[:, pl.ds(0, LANE)]

        def drain_reads(slot, rsem):
            pltpu.make_async_copy(dummy, eslab.at[slot], rsem).wait()
            pltpu.make_async_copy(dummy, rslab.at[slot], rsem).wait()

        def drain_writes(slot, wsem):
            pltpu.make_async_copy(
                dummy, epack.at[slot, :, pl.ds(0, LANE)], wsem).wait()
            pltpu.make_async_copy(
                dummy, rpack.at[slot, :, pl.ds(0, LANE)], wsem).wait()

        def do_slab(t, i, slot, rsem, wsem):
            drain_reads(slot, rsem)

            @pl.when(i > 0)
            def _():
                drain_writes(slot, wsem)

            _transpose_slab(eslab.at[slot], epack.at[slot], parity64, row_half, L)
            _transpose_slab(rslab.at[slot], rpack.at[slot], parity64, row_half, L)
            cb = slab_of(t)
            row = pl.ds(cb * DIM, DIM)
            pltpu.async_copy(
                epack.at[slot, :, pl.ds(0, LANE)], eout_hbm.at[row], wsem)
            pltpu.async_copy(
                rpack.at[slot, :, pl.ds(0, LANE)], rout_hbm.at[row], wsem)

        fire_reads(0, 0, rsem0)
        fire_reads(1, 1, rsem1)

        def step(i, carry):
            t0 = 2 * i
            do_slab(t0, i, 0, rsem0, wsem0)

            @pl.when(t0 + 2 < 2 * STEPS)
            def _():
                fire_reads(t0 + 2, 0, rsem0)

            do_slab(t0 + 1, i, 1, rsem1, wsem1)

            @pl.when(t0 + 3 < 2 * STEPS)
            def _():
                fire_reads(t0 + 3, 1, rsem1)

            return carry

        lax.fori_loop(0, STEPS, step, 0)
        drain_writes(0, wsem0)
        drain_writes(1, wsem1)

    return ka


@functools.cache
def _make_phase_b(B: int, n_rows: int):
    info = plsc.get_sparse_core_info()
    NC, NS, L = info.num_cores, info.num_subcores, info.num_lanes
    NW = NC * NS
    BW = B // NW
    NG = BW // L
    mesh = plsc.VectorSubcoreMesh(core_axis_name="c", subcore_axis_name="s")

    @functools.partial(
        pl.kernel,
        mesh=mesh,
        compiler_params=pltpu.CompilerParams(needs_layout_passes=False),
        out_type=jax.ShapeDtypeStruct((B,), jnp.float32),
        scratch_types=[
            pltpu.VMEM((BW,), jnp.int32),
            pltpu.VMEM((BW,), jnp.int32),
            pltpu.VMEM((BW,), jnp.int32),
            pltpu.VMEM((L, 2 * DIM), jnp.float32),
            pltpu.VMEM((L, 2 * DIM), jnp.float32),
            pltpu.VMEM((L, 2 * DIM), jnp.float32),
            pltpu.VMEM((BW,), jnp.float32),
            pltpu.SemaphoreType.DMA,
        ],
    )
    def kb(hidx_hbm, ridx_hbm, tidx_hbm, ent_hbm, rel_hbm, out_hbm,
           hidx_v, ridx_v, tidx_v, h_v, r_v, t_v, out_v, sem):
        wid = lax.axis_index("s") * NC + lax.axis_index("c")
        base = wid * BW
        pltpu.sync_copy(hidx_hbm.at[pl.ds(base, BW)], hidx_v)
        pltpu.sync_copy(ridx_hbm.at[pl.ds(base, BW)], ridx_v)
        pltpu.sync_copy(tidx_hbm.at[pl.ds(base, BW)], tidx_v)

        lanes = lax.iota(jnp.int32, L)

        def group(g, carry):
            sl = pl.ds(g * L, L)
            his = hidx_v[sl]
            ris = ridx_v[sl]
            tis = tidx_v[sl]
            cps = [
                pltpu.async_copy(ent_hbm.at[his >> 1], h_v, sem),
                pltpu.async_copy(rel_hbm.at[ris >> 1], r_v, sem),
                pltpu.async_copy(ent_hbm.at[tis >> 1], t_v, sem),
            ]
            for c in cps:
                c.wait()

            hoff = (his & 1) * DIM
            roff = (ris & 1) * DIM
            toff = (tis & 1) * DIM
            acc = jnp.zeros((L,), jnp.float32)
            for d in range(DIM):
                h = plsc.load_gather(h_v, [lanes, hoff + d])
                r = plsc.load_gather(r_v, [lanes, roff + d])
                t = plsc.load_gather(t_v, [lanes, toff + d])
                acc = acc + jnp.abs(h + r - t)
            out_v[sl] = GAMMA - acc
            return carry

        lax.fori_loop(0, NG, group, 0)
        pltpu.sync_copy(out_v, out_hbm.at[pl.ds(base, BW)])

    return kb


@jax.jit
def kernel(sample, entity_embedding, relation_embedding):
    B = sample.shape[0]
    n = entity_embedding.shape[0]
    hidx = sample[:, 0]
    ridx = sample[:, 1]
    tidx = sample[:, 2]
    ent_stage, rel_stage = _make_phase_a(n)(entity_embedding.T,
                                            relation_embedding.T)
    score = _make_phase_b(B, ent_stage.shape[0])(
        hidx, ridx, tidx, ent_stage, rel_stage)
    return score.reshape(B, 1)


# speedup vs baseline: 2.6301x; 2.6301x over previous
import functools

import jax
import jax.numpy as jnp
from jax import lax
from jax.experimental import pallas as pl
from jax.experimental.pallas import tpu as pltpu
from jax.experimental.pallas import tpu_sc as plsc

DIM = 64
GAMMA = 12.0
CHUNK = 1024


@functools.cache
def _make_tc_transpose(n_entity: int):
    grid = (n_entity // 2 + CHUNK - 1) // CHUNK
    n_rows = grid * CHUNK
    last_in_block = (n_entity - 1) // CHUNK

    def hi_map(c):
        return (0, jnp.minimum(c + grid, last_in_block))

    def body(elo_ref, ehi_ref, rlo_ref, rhi_ref, eo_ref, ro_ref):
        for lo, hi, dst in ((elo_ref, ehi_ref, eo_ref),
                            (rlo_ref, rhi_ref, ro_ref)):
            dst[:, 0:DIM] = lo[...].T
            dst[:, DIM:2 * DIM] = hi[...].T

    return pl.pallas_call(
        body,
        grid=(grid,),
        in_specs=[
            pl.BlockSpec((DIM, CHUNK), lambda c: (0, c)),
            pl.BlockSpec((DIM, CHUNK), hi_map),
            pl.BlockSpec((DIM, CHUNK), lambda c: (0, c)),
            pl.BlockSpec((DIM, CHUNK), hi_map),
        ],
        out_specs=[
            pl.BlockSpec((CHUNK, 2 * DIM), lambda c: (c, 0)),
            pl.BlockSpec((CHUNK, 2 * DIM), lambda c: (c, 0)),
        ],
        out_shape=[
            jax.ShapeDtypeStruct((n_rows, 2 * DIM), jnp.float32),
            jax.ShapeDtypeStruct((n_rows, 2 * DIM), jnp.float32),
        ],
    )


@functools.cache
def _make_phase_b(B: int, n_rows: int):
    info = plsc.get_sparse_core_info()
    NC, NS, L = info.num_cores, info.num_subcores, info.num_lanes
    NW = NC * NS
    BW = B // NW
    NG = BW // L
    mesh = plsc.VectorSubcoreMesh(core_axis_name="c", subcore_axis_name="s")

    @functools.partial(
        pl.kernel,
        mesh=mesh,
        compiler_params=pltpu.CompilerParams(needs_layout_passes=False),
        out_type=jax.ShapeDtypeStruct((B,), jnp.float32),
        scratch_types=[
            pltpu.VMEM((BW,), jnp.int32),
            pltpu.VMEM((BW,), jnp.int32),
            pltpu.VMEM((BW,), jnp.int32),
            pltpu.VMEM((L, 2 * DIM), jnp.float32),
            pltpu.VMEM((L, 2 * DIM), jnp.float32),
            pltpu.VMEM((L, 2 * DIM), jnp.float32),
            pltpu.VMEM((BW,), jnp.float32),
            pltpu.SemaphoreType.DMA,
        ],
    )
    def kb(hidx_hbm, ridx_hbm, tidx_hbm, ent_hbm, rel_hbm, out_hbm,
           hidx_v, ridx_v, tidx_v, h_v, r_v, t_v, out_v, sem):
        wid = lax.axis_index("s") * NC + lax.axis_index("c")
        base = wid * BW
        pltpu.sync_copy(hidx_hbm.at[pl.ds(base, BW)], hidx_v)
        pltpu.sync_copy(ridx_hbm.at[pl.ds(base, BW)], ridx_v)
        pltpu.sync_copy(tidx_hbm.at[pl.ds(base, BW)], tidx_v)

        lanes = lax.iota(jnp.int32, L)

        def group(g, carry):
            sl = pl.ds(g * L, L)
            his = hidx_v[sl]
            ris = ridx_v[sl]
            tis = tidx_v[sl]
            hrow = jnp.where(his < n_rows, his, his - n_rows)
            rrow = jnp.where(ris < n_rows, ris, ris - n_rows)
            trow = jnp.where(tis < n_rows, tis, tis - n_rows)
            cps = [
                pltpu.async_copy(ent_hbm.at[hrow], h_v, sem),
                pltpu.async_copy(rel_hbm.at[rrow], r_v, sem),
                pltpu.async_copy(ent_hbm.at[trow], t_v, sem),
            ]
            for c in cps:
                c.wait()

            hoff = jnp.where(his < n_rows, 0, DIM)
            roff = jnp.where(ris < n_rows, 0, DIM)
            toff = jnp.where(tis < n_rows, 0, DIM)
            acc = jnp.zeros((L,), jnp.float32)
            for d in range(DIM):
                h = plsc.load_gather(h_v, [lanes, hoff + d])
                r = plsc.load_gather(r_v, [lanes, roff + d])
                t = plsc.load_gather(t_v, [lanes, toff + d])
                acc = acc + jnp.abs(h + r - t)
            out_v[sl] = GAMMA - acc
            return carry

        lax.fori_loop(0, NG, group, 0)
        pltpu.sync_copy(out_v, out_hbm.at[pl.ds(base, BW)])

    return kb


@jax.jit
def kernel(sample, entity_embedding, relation_embedding):
    B = sample.shape[0]
    n = entity_embedding.shape[0]
    hidx = sample[:, 0]
    ridx = sample[:, 1]
    tidx = sample[:, 2]
    ent_t = entity_embedding.T
    rel_t = relation_embedding.T
    ent_stage, rel_stage = _make_tc_transpose(n)(ent_t, ent_t, rel_t, rel_t)
    score = _make_phase_b(B, ent_stage.shape[0])(
        hidx, ridx, tidx, ent_stage, rel_stage)
    return score.reshape(B, 1)
